# baseline (speedup 1.0000x reference)
import functools

import jax
import jax.numpy as jnp
from jax import lax
from jax.experimental import pallas as pl
from jax.experimental.pallas import tpu as pltpu
from jax.experimental.pallas import tpu_sc as plsc

NC = 2
NS = 16
NW = NC * NS
LANE = 16

B = 4096
L = 200
EMB = 64
NLAB = 5
VOCAB = 1000000

BPW = B // NW
NIDX = BPW * L
CA, CB = 104, 96
INV_L = 1.0 / L

FW = 8192
FG = (VOCAB + FW - 1) // FW
VP = FG * FW

TCW = VP // 128 // NW
ICH = 6
NCH = TCW // ICH
CHV = ICH * 128


def _fold_body(w_ref, x_ref, o_ref):
    o_ref[...] = lax.dot_general(
        w_ref[...], x_ref[...],
        (((1,), (0,)), ((), ())),
        preferred_element_type=jnp.float32,
    )


def _fold(w16, table_t):
    return pl.pallas_call(
        _fold_body,
        grid=(FG,),
        in_specs=[
            pl.BlockSpec((LANE, EMB), lambda j: (0, 0)),
            pl.BlockSpec((EMB, FW), lambda j: (0, j)),
        ],
        out_specs=pl.BlockSpec((LANE, FW), lambda j: (0, j)),
        out_shape=jax.ShapeDtypeStruct((LANE, VP), jnp.float32),
    )(w16, table_t)


@functools.partial(
    pl.kernel,
    out_type=jax.ShapeDtypeStruct((VP * LANE,), jnp.float32),
    mesh=plsc.VectorSubcoreMesh(
        core_axis_name="c", subcore_axis_name="s",
        num_cores=NC, num_subcores=NS),
    compiler_params=pltpu.CompilerParams(
        use_tc_tiling_on_sc=True, needs_layout_passes=False),
    scratch_types=[
        pltpu.VMEM((LANE, CHV), jnp.float32),
        pltpu.VMEM((LANE, CHV), jnp.float32),
        pltpu.VMEM((CHV * LANE,), jnp.float32),
        pltpu.VMEM((CHV * LANE,), jnp.float32),
        pltpu.SemaphoreType.DMA,
        pltpu.SemaphoreType.DMA,
        pltpu.SemaphoreType.DMA,
        pltpu.SemaphoreType.DMA,
    ],
)
def _interleave_kernel(y_hbm, out_hbm, buf0, buf1, obuf0, obuf1,
                       sem0, sem1, osem0, osem1):
    wid = lax.axis_index("s") * NC + lax.axis_index("c")
    base = wid * TCW * 128

    def fire(ch, buf, sem):
        col0 = base + (ch % NCH) * CHV
        pltpu.async_copy(y_hbm.at[:, pl.ds(col0, CHV)], buf, sem)

    def wait(buf, sem):
        pltpu.make_async_copy(y_hbm.at[:, pl.ds(0, CHV)], buf, sem).wait()

    fire(0, buf0, sem0)
    fire(1, buf1, sem1)

    iota16 = lax.iota(jnp.int32, LANE) * LANE

    def transpose_chunk(buf, ob):
        unroll = 4
        for r in range(LANE):
            def body(g, idx, r=r):
                for u in range(unroll):
                    c0 = (unroll * g + u) * LANE
                    row = buf[r, pl.ds(c0, LANE)]
                    plsc.store_scatter(ob, [idx + u * LANE * LANE], row)
                return idx + unroll * LANE * LANE

            lax.fori_loop(0, CHV // LANE // unroll, body, iota16 + r)

    def flush(ch, ob, osem):
        col0 = base + ch * CHV
        pltpu.async_copy(ob, out_hbm.at[pl.ds(col0 * LANE, CHV * LANE)], osem)

    def owait(ob, osem):
        pltpu.make_async_copy(out_hbm.at[pl.ds(0, CHV * LANE)], ob, osem).wait()

    wait(buf0, sem0)
    transpose_chunk(buf0, obuf0)
    flush(0, obuf0, osem0)
    fire(2, buf0, sem0)
    wait(buf1, sem1)
    transpose_chunk(buf1, obuf1)
    flush(1, obuf1, osem1)
    fire(3, buf1, sem1)

    def step(t, carry):
        del carry
        c0 = 2 * t
        c1 = c0 + 1
        wait(buf0, sem0)
        owait(obuf0, osem0)
        transpose_chunk(buf0, obuf0)
        flush(c0, obuf0, osem0)
        fire(c0 + 2, buf0, sem0)
        wait(buf1, sem1)
        owait(obuf1, osem1)
        transpose_chunk(buf1, obuf1)
        flush(c1, obuf1, osem1)
        fire(c1 + 2, buf1, sem1)
        return 0

    lax.fori_loop(1, (NCH - 1) // 2, step, 0)
    wait(buf0, sem0)
    owait(obuf0, osem0)
    transpose_chunk(buf0, obuf0)
    flush(NCH - 1, obuf0, osem0)
    wait(buf1, sem1)
    owait(obuf0, osem0)
    owait(obuf1, osem1)


def _accum_chunk(buf, n, acc):
    def body(j, acc):
        out = acc
        for u in range(8):
            out = out + buf[8 * j + u, :]
        return out

    return lax.fori_loop(0, n // 8, body, acc)


@functools.partial(
    pl.kernel,
    out_type=jax.ShapeDtypeStruct((B, LANE), jnp.float32),
    mesh=plsc.VectorSubcoreMesh(
        core_axis_name="c", subcore_axis_name="s",
        num_cores=NC, num_subcores=NS),
    compiler_params=pltpu.CompilerParams(use_tc_tiling_on_sc=False),
    scratch_types=[
        pltpu.VMEM((NIDX,), jnp.int32),
        pltpu.VMEM((BPW, LANE), jnp.float32),
        pltpu.VMEM((LANE,), jnp.float32),
        pltpu.VMEM((CA, LANE), jnp.float32),
        pltpu.VMEM((CB, LANE), jnp.float32),
        pltpu.VMEM((CA, LANE), jnp.float32),
        pltpu.VMEM((CB, LANE), jnp.float32),
        pltpu.SemaphoreType.DMA,
        pltpu.SemaphoreType.DMA,
        pltpu.SemaphoreType.DMA,
        pltpu.SemaphoreType.DMA,
    ],
)
def _pool_kernel(idx_hbm, p_hbm, bias_hbm, out_hbm, idx_v, pooled_v, bias_v,
                 buf_a0, buf_b0, buf_a1, buf_b1,
                 sem_a0, sem_b0, sem_a1, sem_b1):
    wid = lax.axis_index("s") * NC + lax.axis_index("c")

    pltpu.sync_copy(idx_hbm.at[pl.ds(wid * NIDX, NIDX)], idx_v)
    pltpu.sync_copy(bias_hbm, bias_v)
    bias = bias_v[...]

    def fire(row, off, size, buf, sem):
        start = row * L + off
        pltpu.async_copy(p_hbm.at[idx_v.at[pl.ds(start, size)]], buf, sem)

    def wait(size, buf, sem):
        pltpu.make_async_copy(p_hbm.at[pl.ds(0, size)], buf, sem).wait()

    fire(0, 0, CA, buf_a0, sem_a0)
    fire(0, CA, CB, buf_b0, sem_b0)
    fire(1, 0, CA, buf_a1, sem_a1)
    fire(1, CA, CB, buf_b1, sem_b1)

    zero = jnp.zeros((LANE,), jnp.float32)

    def step(t, carry):
        del carry
        r0 = 2 * t
        r1 = r0 + 1
        n0 = (r0 + 2) & (BPW - 1)
        n1 = (r1 + 2) & (BPW - 1)

        wait(CA, buf_a0, sem_a0)
        acc = _accum_chunk(buf_a0, CA, zero)
        fire(n0, 0, CA, buf_a0, sem_a0)
        wait(CB, buf_b0, sem_b0)
        acc = _accum_chunk(buf_b0, CB, acc)
        fire(n0, CA, CB, buf_b0, sem_b0)
        pooled_v[r0, :] = acc + bias

        wait(CA, buf_a1, sem_a1)
        acc = _accum_chunk(buf_a1, CA, zero)
        fire(n1, 0, CA, buf_a1, sem_a1)
        wait(CB, buf_b1, sem_b1)
        acc = _accum_chunk(buf_b1, CB, acc)
        fire(n1, CA, CB, buf_b1, sem_b1)
        pooled_v[r1, :] = acc + bias

        return 0

    lax.fori_loop(0, BPW // 2, step, 0)

    wait(CA, buf_a0, sem_a0)
    wait(CB, buf_b0, sem_b0)
    wait(CA, buf_a1, sem_a1)
    wait(CB, buf_b1, sem_b1)

    pltpu.sync_copy(pooled_v, out_hbm.at[pl.ds(wid * BPW, BPW)])


@jax.jit
def kernel(text, emb_table, fc_w, fc_b):
    table_t = emb_table.T
    w16 = jnp.pad(fc_w * INV_L, ((0, LANE - NLAB), (0, 0)))
    y16 = _fold(w16, table_t)
    flat = _interleave_kernel(y16)
    p16 = flat.reshape(VP, LANE)
    bias16 = jnp.pad(fc_b, (0, LANE - NLAB))
    pooled = _pool_kernel(text.reshape(-1), p16, bias16)
    return pooled[:, :NLAB]


# speedup vs baseline: 1.2748x; 1.2748x over previous
import functools

import jax
import jax.numpy as jnp
from jax import lax
from jax.experimental import pallas as pl
from jax.experimental.pallas import tpu as pltpu
from jax.experimental.pallas import tpu_sc as plsc

NC = 2
NS = 16
NW = NC * NS
LANE = 16

B = 4096
L = 200
EMB = 64
NLAB = 5
VOCAB = 1000000
NPLANE = 8

BPW = B // NW
NIDX = BPW * L
CA, CB = 104, 96
INV_L = 1.0 / L

FW = 8192
FG = (VOCAB + FW - 1) // FW
VP = FG * FW

TCW = VP // 128 // NW
ICH = 6
NCH = TCW // ICH
CHV = ICH * 128


def _fold_body(w_ref, x_ref, o_ref):
    o_ref[...] = lax.dot_general(
        w_ref[...], x_ref[...],
        (((1,), (0,)), ((), ())),
        preferred_element_type=jnp.float32,
    )


def _fold(w8, table_t):
    return pl.pallas_call(
        _fold_body,
        grid=(FG,),
        in_specs=[
            pl.BlockSpec((NPLANE, EMB), lambda j: (0, 0)),
            pl.BlockSpec((EMB, FW), lambda j: (0, j)),
        ],
        out_specs=pl.BlockSpec((NPLANE, FW), lambda j: (0, j)),
        out_shape=jax.ShapeDtypeStruct((NPLANE, VP), jnp.float32),
    )(w8, table_t)


@functools.partial(
    pl.kernel,
    out_type=jax.ShapeDtypeStruct((VP * LANE,), jnp.float32),
    mesh=plsc.VectorSubcoreMesh(
        core_axis_name="c", subcore_axis_name="s",
        num_cores=NC, num_subcores=NS),
    compiler_params=pltpu.CompilerParams(
        use_tc_tiling_on_sc=True, needs_layout_passes=False),
    scratch_types=[
        pltpu.VMEM((NPLANE, CHV), jnp.float32),
        pltpu.VMEM((NPLANE, CHV), jnp.float32),
        pltpu.VMEM((CHV * LANE,), jnp.float32),
        pltpu.VMEM((CHV * LANE,), jnp.float32),
        pltpu.SemaphoreType.DMA,
        pltpu.SemaphoreType.DMA,
        pltpu.SemaphoreType.DMA,
        pltpu.SemaphoreType.DMA,
    ],
)
def _interleave_kernel(y_hbm, out_hbm, buf0, buf1, obuf0, obuf1,
                       sem0, sem1, osem0, osem1):
    wid = lax.axis_index("s") * NC + lax.axis_index("c")
    base = wid * TCW * 128

    def fire(ch, buf, sem):
        col0 = base + (ch % NCH) * CHV
        pltpu.async_copy(y_hbm.at[:, pl.ds(col0, CHV)], buf, sem)

    def wait(buf, sem):
        pltpu.make_async_copy(y_hbm.at[:, pl.ds(0, CHV)], buf, sem).wait()

    fire(0, buf0, sem0)
    fire(1, buf1, sem1)

    zvec = jnp.zeros((LANE,), jnp.float32)

    def zfill(i, carry):
        del carry
        obuf0[pl.ds(i * LANE, LANE)] = zvec
        obuf1[pl.ds(i * LANE, LANE)] = zvec
        return 0

    lax.fori_loop(0, CHV * LANE // LANE, zfill, 0)

    iota16 = lax.iota(jnp.int32, LANE) * LANE

    def transpose_chunk(buf, ob):
        unroll = 4
        for r in range(NPLANE):
            def body(g, idx, r=r):
                for u in range(unroll):
                    c0 = (unroll * g + u) * LANE
                    row = buf[r, pl.ds(c0, LANE)]
                    plsc.store_scatter(ob, [idx + u * LANE * LANE], row)
                return idx + unroll * LANE * LANE

            lax.fori_loop(0, CHV // LANE // unroll, body, iota16 + r)

    def flush(ch, ob, osem):
        col0 = base + ch * CHV
        pltpu.async_copy(ob, out_hbm.at[pl.ds(col0 * LANE, CHV * LANE)], osem)

    def owait(ob, osem):
        pltpu.make_async_copy(out_hbm.at[pl.ds(0, CHV * LANE)], ob, osem).wait()

    wait(buf0, sem0)
    transpose_chunk(buf0, obuf0)
    flush(0, obuf0, osem0)
    fire(2, buf0, sem0)
    wait(buf1, sem1)
    transpose_chunk(buf1, obuf1)
    flush(1, obuf1, osem1)
    fire(3, buf1, sem1)

    def step(t, carry):
        del carry
        c0 = 2 * t
        c1 = c0 + 1
        wait(buf0, sem0)
        owait(obuf0, osem0)
        transpose_chunk(buf0, obuf0)
        flush(c0, obuf0, osem0)
        fire(c0 + 2, buf0, sem0)
        wait(buf1, sem1)
        owait(obuf1, osem1)
        transpose_chunk(buf1, obuf1)
        flush(c1, obuf1, osem1)
        fire(c1 + 2, buf1, sem1)
        return 0

    lax.fori_loop(1, (NCH - 1) // 2, step, 0)
    wait(buf0, sem0)
    owait(obuf0, osem0)
    transpose_chunk(buf0, obuf0)
    flush(NCH - 1, obuf0, osem0)
    wait(buf1, sem1)
    owait(obuf0, osem0)
    owait(obuf1, osem1)


def _accum_chunk(buf, n, acc):
    def body(j, acc):
        out = acc
        for u in range(8):
            out = out + buf[8 * j + u, :]
        return out

    return lax.fori_loop(0, n // 8, body, acc)


@functools.partial(
    pl.kernel,
    out_type=jax.ShapeDtypeStruct((B, LANE), jnp.float32),
    mesh=plsc.VectorSubcoreMesh(
        core_axis_name="c", subcore_axis_name="s",
        num_cores=NC, num_subcores=NS),
    compiler_params=pltpu.CompilerParams(use_tc_tiling_on_sc=False),
    scratch_types=[
        pltpu.VMEM((NIDX,), jnp.int32),
        pltpu.VMEM((BPW, LANE), jnp.float32),
        pltpu.VMEM((LANE,), jnp.float32),
        pltpu.VMEM((CA, LANE), jnp.float32),
        pltpu.VMEM((CB, LANE), jnp.float32),
        pltpu.VMEM((CA, LANE), jnp.float32),
        pltpu.VMEM((CB, LANE), jnp.float32),
        pltpu.SemaphoreType.DMA,
        pltpu.SemaphoreType.DMA,
        pltpu.SemaphoreType.DMA,
        pltpu.SemaphoreType.DMA,
    ],
)
def _pool_kernel(idx_hbm, p_hbm, bias_hbm, out_hbm, idx_v, pooled_v, bias_v,
                 buf_a0, buf_b0, buf_a1, buf_b1,
                 sem_a0, sem_b0, sem_a1, sem_b1):
    wid = lax.axis_index("s") * NC + lax.axis_index("c")

    pltpu.sync_copy(idx_hbm.at[pl.ds(wid * NIDX, NIDX)], idx_v)
    pltpu.sync_copy(bias_hbm, bias_v)
    bias = bias_v[...]

    def fire(row, off, size, buf, sem):
        start = row * L + off
        pltpu.async_copy(p_hbm.at[idx_v.at[pl.ds(start, size)]], buf, sem)

    def wait(size, buf, sem):
        pltpu.make_async_copy(p_hbm.at[pl.ds(0, size)], buf, sem).wait()

    fire(0, 0, CA, buf_a0, sem_a0)
    fire(0, CA, CB, buf_b0, sem_b0)
    fire(1, 0, CA, buf_a1, sem_a1)
    fire(1, CA, CB, buf_b1, sem_b1)

    zero = jnp.zeros((LANE,), jnp.float32)

    def step(t, carry):
        del carry
        r0 = 2 * t
        r1 = r0 + 1
        n0 = (r0 + 2) & (BPW - 1)
        n1 = (r1 + 2) & (BPW - 1)

        wait(CA, buf_a0, sem_a0)
        acc = _accum_chunk(buf_a0, CA, zero)
        fire(n0, 0, CA, buf_a0, sem_a0)
        wait(CB, buf_b0, sem_b0)
        acc = _accum_chunk(buf_b0, CB, acc)
        fire(n0, CA, CB, buf_b0, sem_b0)
        pooled_v[r0, :] = acc + bias

        wait(CA, buf_a1, sem_a1)
        acc = _accum_chunk(buf_a1, CA, zero)
        fire(n1, 0, CA, buf_a1, sem_a1)
        wait(CB, buf_b1, sem_b1)
        acc = _accum_chunk(buf_b1, CB, acc)
        fire(n1, CA, CB, buf_b1, sem_b1)
        pooled_v[r1, :] = acc + bias

        return 0

    lax.fori_loop(0, BPW // 2, step, 0)

    wait(CA, buf_a0, sem_a0)
    wait(CB, buf_b0, sem_b0)
    wait(CA, buf_a1, sem_a1)
    wait(CB, buf_b1, sem_b1)

    pltpu.sync_copy(pooled_v, out_hbm.at[pl.ds(wid * BPW, BPW)])


@jax.jit
def kernel(text, emb_table, fc_w, fc_b):
    table_t = emb_table.T
    w8 = jnp.pad(fc_w * INV_L, ((0, NPLANE - NLAB), (0, 0)))
    y8 = _fold(w8, table_t)
    flat = _interleave_kernel(y8)
    p16 = flat.reshape(VP, LANE)
    bias16 = jnp.pad(fc_b, (0, LANE - NLAB))
    pooled = _pool_kernel(text.reshape(-1), p16, bias16)
    return pooled[:, :NLAB]
